# Initial kernel scaffold; baseline (speedup 1.0000x reference)
import jax
import jax.numpy as jnp
from jax.experimental import pallas as pl


def kernel(x0, x2, x3, emb_word, emb_bi, emb_tri, W1, b1, W2, b2):
    raise NotImplementedError("write your pallas kernel here")


# speedup vs baseline: 5.6876x; 5.6876x over previous
import functools

import jax
import jax.numpy as jnp
from jax import lax
from jax.experimental import pallas as pl
from jax.experimental.pallas import tpu as pltpu
from jax.experimental.pallas import tpu_sc as plsc

B = 1024
L = 50
D = 768
NW = 32
PER_W = B // NW
HALF = D // 2
NREG = HALF // 16


def _reduce_rows(buf, out_ref):
    for h in range(2):
        base = h * HALF

        def rbody(r, accs):
            return tuple(
                a + buf[r, pl.ds(base + j * 16, 16)] for j, a in enumerate(accs)
            )

        accs = lax.fori_loop(
            0, L, rbody,
            tuple(jnp.zeros((16,), jnp.float32) for _ in range(NREG)),
        )
        for j in range(NREG):
            out_ref[pl.ds(base + j * 16, 16)] = accs[j]


def _sc_pool(x0, x2, x3, emb_word, emb_bi, emb_tri):
    mesh = plsc.VectorSubcoreMesh(core_axis_name="c", subcore_axis_name="s")

    @functools.partial(
        pl.kernel,
        mesh=mesh,
        out_type=(
            jax.ShapeDtypeStruct((B, D), jnp.float32),
            jax.ShapeDtypeStruct((B, D), jnp.float32),
            jax.ShapeDtypeStruct((B, D), jnp.float32),
        ),
        scratch_types=[
            pltpu.VMEM((PER_W, L), jnp.int32),
            pltpu.VMEM((L, D), jnp.float32),
            pltpu.VMEM((L, D), jnp.float32),
            pltpu.VMEM((D,), jnp.float32),
            pltpu.VMEM((D,), jnp.float32),
            pltpu.SemaphoreType.DMA,
            pltpu.SemaphoreType.DMA,
        ],
    )
    def k(x0h, x2h, x3h, tw, tb, tt, o0, o1, o2,
          idx_v, buf0, buf1, row0, row1, sem0, sem1):
        cid = lax.axis_index("c")
        sid = lax.axis_index("s")
        wid = sid * 2 + cid
        base = wid * PER_W

        for xh, th, oh in ((x0h, tw, o0), (x2h, tb, o1), (x3h, tt, o2)):
            pltpu.sync_copy(xh.at[pl.ds(base, PER_W)], idx_v)
            pltpu.async_copy(th.at[idx_v.at[0]], buf0, sem0)

            def step(i, _):
                e0 = 2 * i
                e1 = e0 + 1
                pltpu.async_copy(th.at[idx_v.at[e1]], buf1, sem1)
                pltpu.make_async_copy(th.at[idx_v.at[e0]], buf0, sem0).wait()
                _reduce_rows(buf0, row0)
                pltpu.sync_copy(row0, oh.at[base + e0])

                @pl.when(i < PER_W // 2 - 1)
                def _():
                    pltpu.async_copy(th.at[idx_v.at[e0 + 2]], buf0, sem0)

                pltpu.make_async_copy(th.at[idx_v.at[e1]], buf1, sem1).wait()
                _reduce_rows(buf1, row1)
                pltpu.sync_copy(row1, oh.at[base + e1])
                return 0

            lax.fori_loop(0, PER_W // 2, step, 0)

    return k(x0, x2, x3, emb_word, emb_bi, emb_tri)


def _mlp_body(p0, p1, p2, wa, wb, wc, b1r, w2r, b2r, outr):
    h = jnp.dot(p0[...], wa[...], preferred_element_type=jnp.float32)
    h = h + jnp.dot(p1[...], wb[...], preferred_element_type=jnp.float32)
    h = h + jnp.dot(p2[...], wc[...], preferred_element_type=jnp.float32)
    h = h * (1.0 / L) + b1r[...]
    h = jnp.maximum(h, 0.0)
    outr[...] = jnp.dot(h, w2r[...], preferred_element_type=jnp.float32) + b2r[...]


def kernel(x0, x2, x3, emb_word, emb_bi, emb_tri, W1, b1, W2, b2):
    x0 = x0.astype(jnp.int32)
    x2 = x2.astype(jnp.int32)
    x3 = x3.astype(jnp.int32)

    p0, p1, p2 = _sc_pool(x0, x2, x3, emb_word, emb_bi, emb_tri)

    w1t = W1.T
    wa = w1t[:D]
    wb = w1t[D:2 * D]
    wc = w1t[2 * D:]
    H = W1.shape[0]
    C = W2.shape[0]
    CP = 128
    w2p = jnp.zeros((H, CP), jnp.float32).at[:, :C].set(W2.T)
    b2p = jnp.zeros((1, CP), jnp.float32).at[:, :C].set(b2)
    b1r = b1.reshape(1, H)

    out = pl.pallas_call(
        _mlp_body,
        out_shape=jax.ShapeDtypeStruct((B, CP), jnp.float32),
    )(p0, p1, p2, wa, wb, wc, b1r, w2p, b2p)
    return out[:, :C]
